# baseline (speedup 1.0000x reference)
import functools

import jax
import jax.numpy as jnp
from jax import lax
from jax.experimental import pallas as pl
from jax.experimental.pallas import tpu as pltpu
from jax.experimental.pallas import tpu_sc as plsc

NC = 2
NS = 16
NW = NC * NS
NBUF = 8
LOOK = 4


def _sc_kernel(B, L, E, K, V, D, EPT):
    rows_w = B // NW
    n_chunks = rows_w
    assert n_chunks % NBUF == 0 and n_chunks >= 2 * NBUF
    E_pad = ((E + EPT - 1) // EPT) * EPT
    half = EPT // 2
    mesh = plsc.VectorSubcoreMesh(
        core_axis_name="c", subcore_axis_name="s",
        num_cores=NC, num_subcores=NS)

    @functools.partial(
        pl.kernel,
        mesh=mesh,
        out_type=[
            jax.ShapeDtypeStruct((B, L, D), jnp.float32),
            jax.ShapeDtypeStruct((1, E_pad, D), jnp.float32),
        ],
        scratch_types=[
            pltpu.VMEM((half * K,), jnp.int32),
            pltpu.VMEM((half * K, D), jnp.float32),
            pltpu.VMEM((rows_w, L), jnp.int32),
            pltpu.VMEM((NBUF, 1, L, D), jnp.float32),
            pltpu.VMEM((EPT, D), jnp.float32),
            pltpu.SemaphoreType.DMA,
            pltpu.SemaphoreType.DMA((NBUF,)),
            pltpu.SemaphoreType.DMA((NBUF,)),
        ],
        compiler_params=pltpu.CompilerParams(use_tc_tiling_on_sc=False),
    )
    def k(ids_hbm, e2s_hbm, table_hbm, out_hbm, emb_hbm, eidx_v, s1rows,
          idx_all, rows, emb_v, isem, gsem, wsem):
        c = lax.axis_index("c")
        s = lax.axis_index("s")
        wid = s * NC + c
        wrow = wid * rows_w

        idcopy = pltpu.async_copy(ids_hbm.at[pl.ds(wrow, rows_w)], idx_all,
                                  isem)

        base = jnp.minimum(s * EPT, E - EPT)
        for h in range(2):
            hbase = base + h * half
            pltpu.sync_copy(e2s_hbm.at[pl.ds(hbase * K, half * K)], eidx_v)
            pltpu.async_copy(table_hbm.at[eidx_v], s1rows, gsem.at[0]).wait()

            def env_body(e, _):
                for d in range(D // 16):
                    sl = pl.ds(d * 16, 16)
                    acc = s1rows[e * K, sl]
                    for j in range(1, K):
                        acc = acc + s1rows[e * K + j, sl]
                    emb_v[h * half + e, sl] = acc
                return 0

            lax.fori_loop(0, half, env_body, 0)
        pltpu.sync_copy(emb_v, emb_hbm.at[0, pl.ds(base, EPT)])
        plsc.subcore_barrier()
        idcopy.wait()

        def start_gather(i, b):
            return pltpu.async_copy(
                emb_hbm.at[idx_all.at[pl.ds(i, 1)]], rows.at[b],
                gsem.at[b])

        def wait_gather(i, b):
            pltpu.make_async_copy(
                emb_hbm.at[idx_all.at[pl.ds(i, 1)]], rows.at[b],
                gsem.at[b]).wait()

        def start_write(i, b):
            return pltpu.async_copy(rows.at[b],
                                    out_hbm.at[pl.ds(wrow + i, 1)],
                                    wsem.at[b])

        def wait_write(i, b):
            pltpu.make_async_copy(rows.at[b],
                                  out_hbm.at[pl.ds(wrow + i, 1)],
                                  wsem.at[b]).wait()

        for b in range(LOOK):
            start_gather(b, b)
        for b in range(NBUF):
            i = b
            wait_gather(i, b)
            start_write(i, b)
            if b + LOOK < NBUF:
                start_gather(i + LOOK, b + LOOK)
            else:
                wait_write(i - LOOK, (b + LOOK) % NBUF)
                start_gather(i + LOOK, (b + LOOK) % NBUF)

        def group(g, _):
            for b in range(NBUF):
                i = g * NBUF + b
                wait_gather(i, b)
                start_write(i, b)
                j = i + LOOK
                bj = (b + LOOK) % NBUF
                wait_write(j - NBUF, bj)

                @pl.when(j < n_chunks)
                def _():
                    start_gather(j, bj)

            return 0

        lax.fori_loop(1, n_chunks // NBUF, group, 0)

        for t in range(LOOK):
            i = n_chunks - LOOK + t
            wait_write(i, i % NBUF)

    return k


def kernel(env_ids, env2sketchs, sketch_table):
    B, L = env_ids.shape
    E, K = env2sketchs.shape
    V, D = sketch_table.shape
    ids = env_ids.astype(jnp.int32)
    e2s = env2sketchs.reshape(-1).astype(jnp.int32)
    table = sketch_table.astype(jnp.float32)
    k = _sc_kernel(B, L, E, K, V, D, EPT=64)
    out, _ = k(ids, e2s, table)
    return out
